# baseline (speedup 1.0000x reference)
import functools

import jax
import jax.numpy as jnp
from jax import lax
from jax.experimental import pallas as pl
from jax.experimental.pallas import tpu as pltpu
from jax.experimental.pallas import tpu_sc as plsc

NUM_INDICES = 4096 * 200
DIM = 64
PAD_DIM = 128
LANES = 16
NUM_CORES = 2
NUM_SUBCORES = 16
NUM_WORKERS = NUM_CORES * NUM_SUBCORES
PER_WORKER = NUM_INDICES // NUM_WORKERS
NBUF = 4
CHUNK = 80
NUM_CHUNKS = PER_WORKER // CHUNK
assert NUM_CHUNKS % NBUF == 0 and NUM_CHUNKS // NBUF >= 2

_mesh = plsc.VectorSubcoreMesh(core_axis_name="c", subcore_axis_name="s")


@functools.partial(
    pl.kernel,
    mesh=_mesh,
    out_type=jax.ShapeDtypeStruct((NUM_INDICES, DIM), jnp.float32),
    scratch_types=[
        pltpu.VMEM((PER_WORKER,), jnp.int32),
        [pltpu.VMEM((CHUNK, PAD_DIM), jnp.float32) for _ in range(NBUF)],
        [pltpu.VMEM((CHUNK, DIM), jnp.float32) for _ in range(NBUF)],
        [pltpu.SemaphoreType.DMA for _ in range(NBUF)],
        [pltpu.SemaphoreType.DMA for _ in range(NBUF)],
    ],
)
def _gather_kernel(idx_hbm, table_hbm, out_hbm, idx_v, rows, packed, gsems, ssems):
    wid = lax.axis_index("s") * NUM_CORES + lax.axis_index("c")
    base = wid * PER_WORKER

    def start_gather(g, b):
        pltpu.async_copy(
            table_hbm.at[idx_v.at[pl.ds(g * CHUNK, CHUNK)]], rows[b], gsems[b]
        )

    def wait_gather(b):
        pltpu.make_async_copy(
            table_hbm.at[idx_v.at[pl.ds(0, CHUNK)]], rows[b], gsems[b]
        ).wait()

    def compact(b):
        def rows8(m8, carry):
            m0 = m8 * 8
            for r in range(8):
                for k in range(DIM // LANES):
                    packed[b][m0 + r, pl.ds(k * LANES, LANES)] = rows[b][
                        m0 + r, pl.ds(k * LANES, LANES)
                    ]
            return carry

        lax.fori_loop(0, CHUNK // 8, rows8, 0)

    def start_store(g, b):
        pltpu.async_copy(
            packed[b], out_hbm.at[pl.ds(base + g * CHUNK, CHUNK)], ssems[b]
        )

    def wait_store(b):
        pltpu.make_async_copy(
            packed[b], out_hbm.at[pl.ds(base, CHUNK)], ssems[b]
        ).wait()

    pltpu.sync_copy(idx_hbm.at[pl.ds(base, PER_WORKER)], idx_v)

    for b in range(NBUF):
        start_gather(b, b)
        if b == NBUF - 1:
            wait_gather(0)
            compact(0)
            start_store(0, 0)

    def body(t, carry):
        for b in range(NBUF):
            g = t * NBUF + b
            wait_store(b)
            start_gather(g, b)
            b2 = (b + 1) % NBUF
            wait_gather(b2)
            compact(b2)
            start_store(g - (NBUF - 1), b2)
        return carry

    lax.fori_loop(1, NUM_CHUNKS // NBUF, body, 0)

    for k in range(1, NBUF):
        wait_gather(k)
        compact(k)
        start_store(NUM_CHUNKS - NBUF + k, k)
    for k in range(NBUF):
        wait_store(k)


def kernel(timesteps, table):
    idx = timesteps.reshape(-1).astype(jnp.int32)
    table_p = jnp.pad(table, ((0, 0), (0, PAD_DIM - DIM)))
    out = _gather_kernel(idx, table_p)
    return out.reshape(timesteps.shape + (DIM,))


# speedup vs baseline: 1.4716x; 1.4716x over previous
import functools

import jax
import jax.numpy as jnp
from jax import lax
from jax.experimental.layout import Format, Layout
from jax.experimental import pallas as pl
from jax.experimental.pallas import tpu as pltpu
from jax.experimental.pallas import tpu_sc as plsc

NUM_INDICES = 4096 * 200
DIM = 64
PAD_DIM = 128
LANES = 16
NUM_CORES = 2
NUM_SUBCORES = 16
NUM_WORKERS = NUM_CORES * NUM_SUBCORES
PER_WORKER = NUM_INDICES // NUM_WORKERS
NBUF = 4
CHUNK = 80
NUM_CHUNKS = PER_WORKER // CHUNK
assert NUM_CHUNKS % NBUF == 0 and NUM_CHUNKS // NBUF >= 2

_mesh = plsc.VectorSubcoreMesh(core_axis_name="c", subcore_axis_name="s")


@functools.partial(
    pl.kernel,
    mesh=_mesh,
    out_type=jax.ShapeDtypeStruct((NUM_INDICES, DIM), jnp.float32),
    scratch_types=[
        pltpu.VMEM((PER_WORKER,), jnp.int32),
        [pltpu.VMEM((CHUNK, PAD_DIM), jnp.float32) for _ in range(NBUF)],
        [pltpu.VMEM((CHUNK, DIM), jnp.float32) for _ in range(NBUF)],
        [pltpu.SemaphoreType.DMA for _ in range(NBUF)],
        [pltpu.SemaphoreType.DMA for _ in range(NBUF)],
    ],
)
def _gather_kernel(idx_hbm, table_hbm, out_hbm, idx_v, rows, packed, gsems, ssems):
    wid = lax.axis_index("s") * NUM_CORES + lax.axis_index("c")
    base = wid * PER_WORKER

    def start_gather(g, b):
        pltpu.async_copy(
            table_hbm.at[idx_v.at[pl.ds(g * CHUNK, CHUNK)]], rows[b], gsems[b]
        )

    def wait_gather(b):
        pltpu.make_async_copy(
            table_hbm.at[idx_v.at[pl.ds(0, CHUNK)]], rows[b], gsems[b]
        ).wait()

    def compact(b):
        def rows8(m8, carry):
            m0 = m8 * 8
            for r in range(8):
                for k in range(DIM // LANES):
                    packed[b][m0 + r, pl.ds(k * LANES, LANES)] = rows[b][
                        m0 + r, pl.ds(k * LANES, LANES)
                    ]
            return carry

        lax.fori_loop(0, CHUNK // 8, rows8, 0)

    def start_store(g, b):
        pltpu.async_copy(
            packed[b], out_hbm.at[pl.ds(base + g * CHUNK, CHUNK)], ssems[b]
        )

    def wait_store(b):
        pltpu.make_async_copy(
            packed[b], out_hbm.at[pl.ds(base, CHUNK)], ssems[b]
        ).wait()

    pltpu.sync_copy(idx_hbm.at[pl.ds(base, PER_WORKER)], idx_v)

    for b in range(NBUF):
        start_gather(b, b)
        if b == NBUF - 1:
            wait_gather(0)
            compact(0)
            start_store(0, 0)

    def body(t, carry):
        for b in range(NBUF):
            g = t * NBUF + b
            wait_store(b)
            start_gather(g, b)
            b2 = (b + 1) % NBUF
            wait_gather(b2)
            compact(b2)
            start_store(g - (NBUF - 1), b2)
        return carry

    lax.fori_loop(1, NUM_CHUNKS // NBUF, body, 0)

    for k in range(1, NBUF):
        wait_gather(k)
        compact(k)
        start_store(NUM_CHUNKS - NBUF + k, k)
    for k in range(NBUF):
        wait_store(k)


def kernel(timesteps, table):
    idx = timesteps.reshape(-1).astype(jnp.int32)
    table_p = jnp.pad(table, ((0, 0), (0, PAD_DIM - DIM)))
    out = _gather_kernel(idx, table_p)
    out = out.reshape(timesteps.shape + (DIM,))
    return jax.experimental.layout.with_layout_constraint(out, Layout((0, 1, 2)))
